# baseline (speedup 1.0000x reference)
import jax
import jax.numpy as jnp
from jax import lax
from jax.experimental import pallas as pl
from jax.experimental.pallas import tpu as pltpu
from jax.experimental.pallas import tpu_sc as plsc

B = 4
E = 16
H = 2048
D = 2048

SC_ROWS = 128
TC_ROWS = H - SC_ROWS
RCHUNK = 128
NCHUNK = TC_ROWS // RCHUNK

NSUB = 32
SUB_PER_B = NSUB // B
RSUB = SC_ROWS // SUB_PER_B
LANES = 16


def _tc_reduce_body(x_ref, w_ref, h_ref):
    c = pl.program_id(0)
    xb = x_ref[...]
    s = jnp.sum(xb, axis=2) * (1.0 / D) + jnp.max(xb, axis=2)
    part = lax.dot_general(
        s, w_ref[...], (((1,), (1,)), ((), ())),
        preferred_element_type=jnp.float32)

    @pl.when(c == 0)
    def _():
        h_ref[...] = jnp.zeros_like(h_ref)

    h_ref[...] += part


def _tc_partial(x, W):
    return pl.pallas_call(
        _tc_reduce_body,
        grid=(NCHUNK,),
        in_specs=[
            pl.BlockSpec((B, RCHUNK, D), lambda c: (0, c, 0)),
            pl.BlockSpec((E, RCHUNK), lambda c: (0, c)),
        ],
        out_specs=pl.BlockSpec((B, E), lambda c: (0, 0)),
        out_shape=jax.ShapeDtypeStruct((B, E), jnp.float32),
    )(x, W)


def _sc_reduce_body(x_hbm, wt_hbm, part_hbm, xrows_ref, wt_ref, out_ref):
    nc = plsc.get_sparse_core_info().num_cores
    wid = lax.axis_index("s") * nc + lax.axis_index("c")
    b = wid // SUB_PER_B
    row0 = (H - SC_ROWS) + (wid % SUB_PER_B) * RSUB

    pltpu.sync_copy(x_hbm.at[b, pl.ds(row0, RSUB)], xrows_ref)
    pltpu.sync_copy(wt_hbm.at[pl.ds(row0, RSUB)], wt_ref)

    def row_step(r, h_acc):
        def col_step(i, carry):
            vs, vm = carry
            base = i * (8 * LANES)
            for j in range(8):
                v = xrows_ref[r, pl.ds(base + j * LANES, LANES)]
                vs = vs + v
                vm = jnp.maximum(vm, v)
            return (vs, vm)

        v0 = xrows_ref[r, pl.ds(0, LANES)]
        vs, vm = lax.fori_loop(
            0, D // (8 * LANES) - 1,
            lambda i, c: col_step(i + 1, c),
            col_step(0, (jnp.zeros((LANES,), jnp.float32), v0)),
        )
        s_r = jnp.sum(vs) * (1.0 / D) + jnp.max(vm)
        return h_acc + s_r * wt_ref[r, :]

    h_acc = lax.fori_loop(0, RSUB, row_step, jnp.zeros((LANES,), jnp.float32))
    out_ref[...] = h_acc
    pltpu.sync_copy(out_ref, part_hbm.at[wid])


def _sc_partial(x, Wt):
    f = pl.kernel(
        _sc_reduce_body,
        out_type=jax.ShapeDtypeStruct((NSUB, E), jnp.float32),
        mesh=plsc.VectorSubcoreMesh(core_axis_name="c", subcore_axis_name="s"),
        scratch_types=[
            pltpu.VMEM((RSUB, D), jnp.float32),
            pltpu.VMEM((RSUB, E), jnp.float32),
            pltpu.VMEM((E,), jnp.float32),
        ],
        compiler_params=pltpu.CompilerParams(needs_layout_passes=False),
    )
    return f(x, Wt)


def _combine_body(htc_ref, part_ref, b_ref, g_ref, m_ref):
    h = htc_ref[...] + jnp.sum(part_ref[...], axis=1) + b_ref[...]
    h = jnp.where(h >= 0.0, h, 0.2 * h)
    lanes = lax.broadcasted_iota(jnp.int32, (B, E), 1)
    m1 = jnp.max(h, axis=1, keepdims=True)
    i1 = jnp.min(jnp.where(h == m1, lanes, E), axis=1, keepdims=True)
    first1 = lanes == i1
    h2 = jnp.where(first1, -jnp.inf, h)
    m2 = jnp.max(h2, axis=1, keepdims=True)
    i2 = jnp.min(jnp.where(h2 == m2, lanes, E), axis=1, keepdims=True)
    mask = first1 | (lanes == i2)
    e = jnp.where(mask, jnp.exp(h - m1), 0.0)
    g_ref[...] = e / jnp.sum(e, axis=1, keepdims=True)
    m_ref[...] = jnp.where(mask, 1.0, 0.0)


def _combine(h_tc, part_sc, b):
    return pl.pallas_call(
        _combine_body,
        in_specs=[
            pl.BlockSpec((B, E), lambda: (0, 0)),
            pl.BlockSpec((B, SUB_PER_B, E), lambda: (0, 0, 0)),
            pl.BlockSpec((1, E), lambda: (0, 0)),
        ],
        out_specs=[
            pl.BlockSpec((B, E), lambda: (0, 0)),
            pl.BlockSpec((B, E), lambda: (0, 0)),
        ],
        out_shape=[
            jax.ShapeDtypeStruct((B, E), jnp.float32),
            jax.ShapeDtypeStruct((B, E), jnp.float32),
        ],
    )(h_tc, part_sc, b.reshape(1, E))


def kernel(x, W, b):
    h_tc = _tc_partial(x, W)
    part_sc = _sc_partial(x, W.T.reshape(H, E))
    gating_coeffs, mask = _combine(h_tc, part_sc.reshape(B, SUB_PER_B, E), b)
    return (gating_coeffs, mask)


# speedup vs baseline: 1.0059x; 1.0059x over previous
import jax
import jax.numpy as jnp
from jax import lax
from jax.experimental import pallas as pl
from jax.experimental.pallas import tpu as pltpu
from jax.experimental.pallas import tpu_sc as plsc

B = 4
E = 16
H = 2048
D = 2048

SC_ROWS = 128
TC_ROWS = H - SC_ROWS
RCHUNK = 128
NCHUNK = TC_ROWS // RCHUNK

NSUB = 32
SUB_PER_B = NSUB // B
RSUB = SC_ROWS // SUB_PER_B
LANES = 16


def _tc_reduce_body(x_ref, w_ref, h_ref):
    c = pl.program_id(0)
    xb = x_ref[...]
    s = jnp.sum(xb, axis=2) * (1.0 / D) + jnp.max(xb, axis=2)
    part = lax.dot_general(
        s, w_ref[...], (((1,), (1,)), ((), ())),
        preferred_element_type=jnp.float32)

    @pl.when(c == 0)
    def _():
        h_ref[...] = jnp.zeros_like(h_ref)

    h_ref[...] += part


def _tc_partial(x, W):
    return pl.pallas_call(
        _tc_reduce_body,
        grid=(NCHUNK,),
        in_specs=[
            pl.BlockSpec((B, RCHUNK, D), lambda c: (0, c, 0)),
            pl.BlockSpec((E, RCHUNK), lambda c: (0, c)),
        ],
        out_specs=pl.BlockSpec((B, E), lambda c: (0, 0)),
        out_shape=jax.ShapeDtypeStruct((B, E), jnp.float32),
    )(x, W)


def _sc_reduce_body(x_hbm, wt_hbm, part_hbm, xrows_ref, wt_ref, out_ref):
    nc = plsc.get_sparse_core_info().num_cores
    wid = lax.axis_index("s") * nc + lax.axis_index("c")
    b = wid // SUB_PER_B
    row0 = (H - SC_ROWS) + (wid % SUB_PER_B) * RSUB

    pltpu.sync_copy(x_hbm.at[b, pl.ds(row0, RSUB)], xrows_ref)
    pltpu.sync_copy(wt_hbm.at[pl.ds(row0, RSUB)], wt_ref)

    def row_step(r, h_acc):
        vs = jnp.zeros((LANES,), jnp.float32)
        vm = xrows_ref[r, pl.ds(0, LANES)]
        for j in range(D // LANES):
            v = xrows_ref[r, pl.ds(j * LANES, LANES)]
            vs = vs + v
            vm = jnp.maximum(vm, v)
        s_r = jnp.sum(vs) * (1.0 / D) + jnp.max(vm)
        return h_acc + s_r * wt_ref[r, :]

    h_acc = lax.fori_loop(0, RSUB, row_step, jnp.zeros((LANES,), jnp.float32))
    out_ref[...] = h_acc
    pltpu.sync_copy(out_ref, part_hbm.at[wid])


def _sc_partial(x, Wt):
    f = pl.kernel(
        _sc_reduce_body,
        out_type=jax.ShapeDtypeStruct((NSUB, E), jnp.float32),
        mesh=plsc.VectorSubcoreMesh(core_axis_name="c", subcore_axis_name="s"),
        scratch_types=[
            pltpu.VMEM((RSUB, D), jnp.float32),
            pltpu.VMEM((RSUB, E), jnp.float32),
            pltpu.VMEM((E,), jnp.float32),
        ],
        compiler_params=pltpu.CompilerParams(needs_layout_passes=False),
    )
    return f(x, Wt)


def _combine_body(htc_ref, part_ref, b_ref, g_ref, m_ref):
    h = htc_ref[...] + jnp.sum(part_ref[...], axis=1) + b_ref[...]
    h = jnp.where(h >= 0.0, h, 0.2 * h)
    lanes = lax.broadcasted_iota(jnp.int32, (B, E), 1)
    m1 = jnp.max(h, axis=1, keepdims=True)
    i1 = jnp.min(jnp.where(h == m1, lanes, E), axis=1, keepdims=True)
    first1 = lanes == i1
    h2 = jnp.where(first1, -jnp.inf, h)
    m2 = jnp.max(h2, axis=1, keepdims=True)
    i2 = jnp.min(jnp.where(h2 == m2, lanes, E), axis=1, keepdims=True)
    mask = first1 | (lanes == i2)
    e = jnp.where(mask, jnp.exp(h - m1), 0.0)
    g_ref[...] = e / jnp.sum(e, axis=1, keepdims=True)
    m_ref[...] = jnp.where(mask, 1.0, 0.0)


def _combine(h_tc, part_sc, b):
    return pl.pallas_call(
        _combine_body,
        in_specs=[
            pl.BlockSpec((B, E), lambda: (0, 0)),
            pl.BlockSpec((B, SUB_PER_B, E), lambda: (0, 0, 0)),
            pl.BlockSpec((1, E), lambda: (0, 0)),
        ],
        out_specs=[
            pl.BlockSpec((B, E), lambda: (0, 0)),
            pl.BlockSpec((B, E), lambda: (0, 0)),
        ],
        out_shape=[
            jax.ShapeDtypeStruct((B, E), jnp.float32),
            jax.ShapeDtypeStruct((B, E), jnp.float32),
        ],
    )(h_tc, part_sc, b.reshape(1, E))


def kernel(x, W, b):
    part_sc = _sc_partial(x, W.T.reshape(H, E))
    h_tc = _tc_partial(x, W)
    gating_coeffs, mask = _combine(h_tc, part_sc.reshape(B, SUB_PER_B, E), b)
    return (gating_coeffs, mask)


# speedup vs baseline: 1.1536x; 1.1469x over previous
import jax
import jax.numpy as jnp
from jax import lax
from jax.experimental import pallas as pl
from jax.experimental.pallas import tpu as pltpu
from jax.experimental.pallas import tpu_sc as plsc

B = 4
E = 16
H = 2048
D = 2048
RCHUNK = 256
NCHUNK = H // RCHUNK


def _reduce_body(x_ref, w_ref, b_ref, h_ref):
    c = pl.program_id(0)
    xb = x_ref[...]
    s = jnp.sum(xb, axis=2) * (1.0 / D) + jnp.max(xb, axis=2)
    part = lax.dot_general(
        s, w_ref[...], (((1,), (1,)), ((), ())),
        preferred_element_type=jnp.float32)

    @pl.when(c == 0)
    def _():
        h_ref[...] = jnp.broadcast_to(b_ref[...], (B, E))

    h_ref[...] += part

    @pl.when(c == NCHUNK - 1)
    def _():
        hv = h_ref[...]
        h_ref[...] = jnp.where(hv >= 0.0, hv, 0.2 * hv)


def _gate_logits(x, W, b):
    return pl.pallas_call(
        _reduce_body,
        grid=(NCHUNK,),
        in_specs=[
            pl.BlockSpec((B, RCHUNK, D), lambda c: (0, c, 0)),
            pl.BlockSpec((E, RCHUNK), lambda c: (0, c)),
            pl.BlockSpec((1, E), lambda c: (0, 0)),
        ],
        out_specs=pl.BlockSpec((B, E), lambda c: (0, 0)),
        out_shape=jax.ShapeDtypeStruct((B, E), jnp.float32),
    )(x, W, b.reshape(1, E))


def _route_body(h_hbm, gat_hbm, mask_hbm, hv_ref, gv_ref, mv_ref):
    sid = lax.axis_index("s")

    @pl.when(sid < B)
    def _():
        pltpu.sync_copy(h_hbm.at[sid], hv_ref)
        hv = hv_ref[...]
        m1 = jnp.max(hv)
        is1 = hv == m1
        first1 = is1 & (jnp.cumsum(is1.astype(jnp.int32)) == 1)
        h2 = jnp.where(first1, -jnp.inf, hv)
        m2 = jnp.max(h2)
        is2 = h2 == m2
        first2 = is2 & (jnp.cumsum(is2.astype(jnp.int32)) == 1)
        mask = first1 | first2
        e = jnp.where(mask, jnp.exp(hv - m1), 0.0)
        gv_ref[...] = e / jnp.sum(e)
        mv_ref[...] = jnp.where(mask, 1.0, 0.0)
        pltpu.sync_copy(gv_ref, gat_hbm.at[sid])
        pltpu.sync_copy(mv_ref, mask_hbm.at[sid])


def _route_sc(h):
    f = pl.kernel(
        _route_body,
        out_type=[
            jax.ShapeDtypeStruct((B, E), jnp.float32),
            jax.ShapeDtypeStruct((B, E), jnp.float32),
        ],
        mesh=plsc.VectorSubcoreMesh(
            core_axis_name="c", subcore_axis_name="s", num_cores=1),
        scratch_types=[
            pltpu.VMEM((E,), jnp.float32),
            pltpu.VMEM((E,), jnp.float32),
            pltpu.VMEM((E,), jnp.float32),
        ],
        compiler_params=pltpu.CompilerParams(needs_layout_passes=False),
    )
    return f(h)


def kernel(x, W, b):
    h = _gate_logits(x, W, b)
    gating_coeffs, mask = _route_sc(h)
    return (gating_coeffs, mask)
